# Initial kernel scaffold; baseline (speedup 1.0000x reference)
import jax
import jax.numpy as jnp
from jax.experimental import pallas as pl


def kernel(x, edge_index, edge_attr, W1, b1, W2, b2, W3, b3):
    raise NotImplementedError("write your pallas kernel here")


# speedup vs baseline: 10.2715x; 10.2715x over previous
import functools

import jax
import jax.numpy as jnp
from jax import lax
from jax.experimental import pallas as pl
from jax.experimental.pallas import tpu as pltpu
from jax.experimental.pallas import tpu_sc as plsc

N = 10000
E = 320000
D = 128
NC = 2
NS = 16
NW = NC * NS
EPT = E // NW
C = 125
NCHUNK = EPT // C
NP = 10240
NPT = NP // NS
ZR = 128
BLK = 2000
BLKD = 2048
DEGW = 128


def _sc_mesh():
    return plsc.VectorSubcoreMesh(core_axis_name="c", subcore_axis_name="s",
                                  num_cores=NC, num_subcores=NS)


def _fill(buf, rows, width, value):
    vec = jnp.full((16,), value, jnp.float32)

    def body(r, _):
        for c8 in range(width // 16):
            buf[r, pl.ds(c8 * 16, 16)] = vec
        return 0

    lax.fori_loop(0, rows, body, 0)


def _deg_scratch():
    return [
        pltpu.VMEM((NCHUNK, C), jnp.int32),
        pltpu.VMEM((ZR, DEGW), jnp.float32),
        pltpu.VMEM_SHARED((NP, DEGW), jnp.float32),
    ]


def _deg_kernel_body(dst_hbm, out_hbm, dst_v, buf, acc):
    cid = lax.axis_index("c")
    sid = lax.axis_index("s")
    wid = cid * NS + sid
    pltpu.sync_copy(dst_hbm.at[wid], dst_v)
    _fill(buf, ZR, DEGW, 0.0)
    for b in range(NPT // ZR):
        pltpu.sync_copy(buf, acc.at[pl.ds(sid * NPT + b * ZR, ZR)])
    _fill(buf, C, DEGW, 1.0)
    plsc.subcore_barrier()

    def body(j, _):
        pltpu.sync_copy(buf.at[pl.ds(0, C)], acc.at[dst_v.at[j]], add=True)
        return 0

    lax.fori_loop(0, NCHUNK, body, 0)
    plsc.subcore_barrier()
    pltpu.sync_copy(acc.at[pl.ds(sid * NPT, NPT)],
                    out_hbm.at[cid, pl.ds(sid * NPT, NPT)])


_deg_kernel = pl.kernel(
    _deg_kernel_body,
    out_type=jax.ShapeDtypeStruct((NC, NP, DEGW), jnp.float32),
    mesh=_sc_mesh(),
    scratch_types=_deg_scratch(),
)


def _hop_scratch():
    return [
        pltpu.VMEM((NCHUNK, C), jnp.int32),
        pltpu.VMEM((NCHUNK, C), jnp.int32),
        pltpu.VMEM((ZR, D), jnp.float32),
        pltpu.VMEM_SHARED((NP, D), jnp.float32),
        pltpu.SemaphoreType.DMA,
    ]


def _hop_kernel_body(u_hbm, src_hbm, dst_hbm, out_hbm, src_v, dst_v, buf, acc, sem):
    cid = lax.axis_index("c")
    sid = lax.axis_index("s")
    wid = cid * NS + sid
    pltpu.sync_copy(src_hbm.at[wid], src_v)
    pltpu.sync_copy(dst_hbm.at[wid], dst_v)
    _fill(buf, ZR, D, 0.0)
    for b in range(NPT // ZR):
        pltpu.sync_copy(buf, acc.at[pl.ds(sid * NPT + b * ZR, ZR)])
    plsc.subcore_barrier()

    def body(j, _):
        pltpu.async_copy(u_hbm.at[src_v.at[j]], buf.at[pl.ds(0, C)], sem).wait()
        pltpu.sync_copy(buf.at[pl.ds(0, C)], acc.at[dst_v.at[j]], add=True)
        return 0

    lax.fori_loop(0, NCHUNK, body, 0)
    plsc.subcore_barrier()
    pltpu.sync_copy(acc.at[pl.ds(sid * NPT, NPT)],
                    out_hbm.at[cid, pl.ds(sid * NPT, NPT)])


_hop_kernel = pl.kernel(
    _hop_kernel_body,
    out_type=jax.ShapeDtypeStruct((NC, NP, D), jnp.float32),
    mesh=_sc_mesh(),
    scratch_types=_hop_scratch(),
)


def _dinv_call(degp):
    def body(d_ref, o_ref):
        d = d_ref[0] + d_ref[1]
        d0 = d[:, 0:1]
        dinv = jnp.where(d0 > 0, lax.rsqrt(d0), 0.0)
        o_ref[...] = jnp.broadcast_to(dinv, (BLKD, D))

    return pl.pallas_call(
        body,
        grid=(NP // BLKD,),
        in_specs=[pl.BlockSpec((NC, BLKD, DEGW), lambda i: (0, i, 0))],
        out_specs=pl.BlockSpec((BLKD, D), lambda i: (i, 0)),
        out_shape=jax.ShapeDtypeStruct((NP, D), jnp.float32),
    )(degp)


def _start_call(h, W0, b, dinv_bc):
    dout = W0.shape[1]

    def body(h_ref, w_ref, b_ref, dv_ref, acc_ref, u_ref):
        hh = h_ref[...]
        acc_ref[...] = (
            jnp.dot(hh, w_ref[...], preferred_element_type=jnp.float32)
            + b_ref[...]
        )
        u_ref[...] = dv_ref[...] * hh

    return pl.pallas_call(
        body,
        grid=(N // BLK,),
        in_specs=[
            pl.BlockSpec((BLK, D), lambda i: (i, 0)),
            pl.BlockSpec((D, dout), lambda i: (0, 0)),
            pl.BlockSpec((1, dout), lambda i: (0, 0)),
            pl.BlockSpec((BLK, D), lambda i: (i, 0)),
        ],
        out_specs=[
            pl.BlockSpec((BLK, dout), lambda i: (i, 0)),
            pl.BlockSpec((BLK, D), lambda i: (i, 0)),
        ],
        out_shape=[
            jax.ShapeDtypeStruct((N, dout), jnp.float32),
            jax.ShapeDtypeStruct((N, D), jnp.float32),
        ],
    )(h, W0, b.reshape(1, dout), dinv_bc)


def _mid_call(sp, dinv_bc, Wk, acc):
    dout = Wk.shape[1]

    def body(sp_ref, dv_ref, w_ref, acc_ref, accO_ref, u_ref):
        s = sp_ref[0] + sp_ref[1]
        dv = dv_ref[...]
        t = dv * s
        accO_ref[...] = acc_ref[...] + jnp.dot(
            t, w_ref[...], preferred_element_type=jnp.float32)
        u_ref[...] = dv * t

    return pl.pallas_call(
        body,
        grid=(N // BLK,),
        in_specs=[
            pl.BlockSpec((NC, BLK, D), lambda i: (0, i, 0)),
            pl.BlockSpec((BLK, D), lambda i: (i, 0)),
            pl.BlockSpec((D, dout), lambda i: (0, 0)),
            pl.BlockSpec((BLK, dout), lambda i: (i, 0)),
        ],
        out_specs=[
            pl.BlockSpec((BLK, dout), lambda i: (i, 0)),
            pl.BlockSpec((BLK, D), lambda i: (i, 0)),
        ],
        out_shape=[
            jax.ShapeDtypeStruct((N, dout), jnp.float32),
            jax.ShapeDtypeStruct((N, D), jnp.float32),
        ],
        input_output_aliases={3: 0},
    )(sp, dinv_bc, Wk, acc)


def _final_call(sp, dinv_bc, Wk, acc, relu):
    dout = Wk.shape[1]

    def body(sp_ref, dv_ref, w_ref, acc_ref, o_ref):
        s = sp_ref[0] + sp_ref[1]
        t = dv_ref[...] * s
        o = acc_ref[...] + jnp.dot(
            t, w_ref[...], preferred_element_type=jnp.float32)
        o_ref[...] = jnp.maximum(o, 0.0) if relu else o

    return pl.pallas_call(
        body,
        grid=(N // BLK,),
        in_specs=[
            pl.BlockSpec((NC, BLK, D), lambda i: (0, i, 0)),
            pl.BlockSpec((BLK, D), lambda i: (i, 0)),
            pl.BlockSpec((D, dout), lambda i: (0, 0)),
            pl.BlockSpec((BLK, dout), lambda i: (i, 0)),
        ],
        out_specs=pl.BlockSpec((BLK, dout), lambda i: (i, 0)),
        out_shape=jax.ShapeDtypeStruct((N, dout), jnp.float32),
        input_output_aliases={3: 0},
    )(sp, dinv_bc, Wk, acc)


def kernel(x, edge_index, edge_attr, W1, b1, W2, b2, W3, b3):
    del edge_attr
    src = edge_index[0].reshape(NW, NCHUNK, C)
    dst = edge_index[1].reshape(NW, NCHUNK, C)

    degp = _deg_kernel(dst)
    dinv_bc = _dinv_call(degp)

    h = x
    for W, b, relu in ((W1, b1, True), (W2, b2, True), (W3, b3, False)):
        acc, u = _start_call(h, W[0], b, dinv_bc)
        for k in range(1, 4):
            sp = _hop_kernel(u, src, dst)
            if k < 3:
                acc, u = _mid_call(sp, dinv_bc, W[k], acc)
            else:
                h = _final_call(sp, dinv_bc, W[k], acc, relu)
    return h


# speedup vs baseline: 11.1337x; 1.0839x over previous
import functools

import jax
import jax.numpy as jnp
from jax import lax
from jax.experimental import pallas as pl
from jax.experimental.pallas import tpu as pltpu
from jax.experimental.pallas import tpu_sc as plsc

N = 10000
E = 320000
D = 128
NC = 2
NS = 16
NW = NC * NS
EPT = E // NW
C = 80
NCHUNK = EPT // C
NP = 10240
NPT = NP // NS
BLK = 2000
BLKD = 2048
DEGW = 128


def _sc_mesh():
    return plsc.VectorSubcoreMesh(core_axis_name="c", subcore_axis_name="s",
                                  num_cores=NC, num_subcores=NS)


def _fill(buf, rows, width, value):
    vec = jnp.full((16,), value, jnp.float32)

    def body(r, _):
        for c8 in range(width // 16):
            buf[r, pl.ds(c8 * 16, 16)] = vec
        return 0

    lax.fori_loop(0, rows, body, 0)


def _deg_scratch():
    return [
        pltpu.VMEM((NCHUNK, C), jnp.int32),
        pltpu.VMEM((C, DEGW), jnp.float32),
        pltpu.VMEM_SHARED((NP, DEGW), jnp.float32),
    ]


def _deg_kernel_body(dst_hbm, zeros_hbm, out_hbm, dst_v, buf, acc):
    cid = lax.axis_index("c")
    sid = lax.axis_index("s")
    wid = cid * NS + sid
    pltpu.sync_copy(dst_hbm.at[wid], dst_v)
    pltpu.sync_copy(zeros_hbm, acc.at[pl.ds(sid * NPT, NPT)])
    _fill(buf, C, DEGW, 1.0)
    plsc.subcore_barrier()

    def body(j, _):
        pltpu.sync_copy(buf, acc.at[dst_v.at[j]], add=True)
        return 0

    lax.fori_loop(0, NCHUNK, body, 0)
    plsc.subcore_barrier()
    pltpu.sync_copy(acc.at[pl.ds(sid * NPT, NPT)],
                    out_hbm.at[cid, pl.ds(sid * NPT, NPT)])


_deg_kernel = pl.kernel(
    _deg_kernel_body,
    out_type=jax.ShapeDtypeStruct((NC, NP, DEGW), jnp.float32),
    mesh=_sc_mesh(),
    scratch_types=_deg_scratch(),
)


def _hop_scratch():
    return [
        pltpu.VMEM((EPT,), jnp.int32),
        pltpu.VMEM((NCHUNK, C), jnp.int32),
        pltpu.VMEM((C, D), jnp.float32),
        pltpu.VMEM((C, D), jnp.float32),
        pltpu.VMEM_SHARED((NP, D), jnp.float32),
        pltpu.SemaphoreType.DMA,
        pltpu.SemaphoreType.DMA,
    ]


def _hop_kernel_body(u_hbm, src_hbm, dst_hbm, zeros_hbm, out_hbm,
                     src_v, dst_v, buf_a, buf_b, acc, sem_a, sem_b):
    cid = lax.axis_index("c")
    sid = lax.axis_index("s")
    wid = cid * NS + sid
    pltpu.sync_copy(src_hbm.at[wid], src_v)
    pltpu.sync_copy(dst_hbm.at[wid], dst_v)
    pltpu.sync_copy(zeros_hbm, acc.at[pl.ds(sid * NPT, NPT)])
    plsc.subcore_barrier()

    def gather(j, buf, sem):
        return pltpu.async_copy(u_hbm.at[src_v.at[pl.ds(j * C, C)]], buf, sem)

    def scatter(j, buf):
        pltpu.sync_copy(buf, acc.at[dst_v.at[j]], add=True)

    gather(0, buf_a, sem_a)

    def pair(k, _):
        j0 = 2 * k
        pltpu.make_async_copy(
            u_hbm.at[src_v.at[pl.ds(j0 * C, C)]], buf_a, sem_a).wait()
        gather(j0 + 1, buf_b, sem_b)
        scatter(j0, buf_a)
        pltpu.make_async_copy(
            u_hbm.at[src_v.at[pl.ds((j0 + 1) * C, C)]], buf_b, sem_b).wait()

        @pl.when(j0 + 2 < NCHUNK)
        def _():
            gather(j0 + 2, buf_a, sem_a)

        scatter(j0 + 1, buf_b)
        return 0

    lax.fori_loop(0, NCHUNK // 2, pair, 0)
    pltpu.make_async_copy(
        u_hbm.at[src_v.at[pl.ds((NCHUNK - 1) * C, C)]], buf_a, sem_a).wait()
    scatter(NCHUNK - 1, buf_a)
    plsc.subcore_barrier()
    pltpu.sync_copy(acc.at[pl.ds(sid * NPT, NPT)],
                    out_hbm.at[cid, pl.ds(sid * NPT, NPT)])


_hop_kernel = pl.kernel(
    _hop_kernel_body,
    out_type=jax.ShapeDtypeStruct((NC, NP, D), jnp.float32),
    mesh=_sc_mesh(),
    scratch_types=_hop_scratch(),
)


def _dinv_call(degp):
    def body(d_ref, o_ref):
        d = d_ref[0] + d_ref[1]
        d0 = d[:, 0:1]
        dinv = jnp.where(d0 > 0, lax.rsqrt(d0), 0.0)
        o_ref[...] = jnp.broadcast_to(dinv, (BLKD, D))

    return pl.pallas_call(
        body,
        grid=(NP // BLKD,),
        in_specs=[pl.BlockSpec((NC, BLKD, DEGW), lambda i: (0, i, 0))],
        out_specs=pl.BlockSpec((BLKD, D), lambda i: (i, 0)),
        out_shape=jax.ShapeDtypeStruct((NP, D), jnp.float32),
    )(degp)


def _start_call(h, W0, b, dinv_bc):
    dout = W0.shape[1]

    def body(h_ref, w_ref, b_ref, dv_ref, acc_ref, u_ref):
        hh = h_ref[...]
        acc_ref[...] = (
            jnp.dot(hh, w_ref[...], preferred_element_type=jnp.float32)
            + b_ref[...]
        )
        u_ref[...] = dv_ref[...] * hh

    return pl.pallas_call(
        body,
        grid=(N // BLK,),
        in_specs=[
            pl.BlockSpec((BLK, D), lambda i: (i, 0)),
            pl.BlockSpec((D, dout), lambda i: (0, 0)),
            pl.BlockSpec((1, dout), lambda i: (0, 0)),
            pl.BlockSpec((BLK, D), lambda i: (i, 0)),
        ],
        out_specs=[
            pl.BlockSpec((BLK, dout), lambda i: (i, 0)),
            pl.BlockSpec((BLK, D), lambda i: (i, 0)),
        ],
        out_shape=[
            jax.ShapeDtypeStruct((N, dout), jnp.float32),
            jax.ShapeDtypeStruct((N, D), jnp.float32),
        ],
    )(h, W0, b.reshape(1, dout), dinv_bc)


def _mid_call(sp, dinv_bc, Wk, acc):
    dout = Wk.shape[1]

    def body(sp_ref, dv_ref, w_ref, acc_ref, accO_ref, u_ref):
        s = sp_ref[0] + sp_ref[1]
        dv = dv_ref[...]
        t = dv * s
        accO_ref[...] = acc_ref[...] + jnp.dot(
            t, w_ref[...], preferred_element_type=jnp.float32)
        u_ref[...] = dv * t

    return pl.pallas_call(
        body,
        grid=(N // BLK,),
        in_specs=[
            pl.BlockSpec((NC, BLK, D), lambda i: (0, i, 0)),
            pl.BlockSpec((BLK, D), lambda i: (i, 0)),
            pl.BlockSpec((D, dout), lambda i: (0, 0)),
            pl.BlockSpec((BLK, dout), lambda i: (i, 0)),
        ],
        out_specs=[
            pl.BlockSpec((BLK, dout), lambda i: (i, 0)),
            pl.BlockSpec((BLK, D), lambda i: (i, 0)),
        ],
        out_shape=[
            jax.ShapeDtypeStruct((N, dout), jnp.float32),
            jax.ShapeDtypeStruct((N, D), jnp.float32),
        ],
        input_output_aliases={3: 0},
    )(sp, dinv_bc, Wk, acc)


def _final_call(sp, dinv_bc, Wk, acc, relu):
    dout = Wk.shape[1]

    def body(sp_ref, dv_ref, w_ref, acc_ref, o_ref):
        s = sp_ref[0] + sp_ref[1]
        t = dv_ref[...] * s
        o = acc_ref[...] + jnp.dot(
            t, w_ref[...], preferred_element_type=jnp.float32)
        o_ref[...] = jnp.maximum(o, 0.0) if relu else o

    return pl.pallas_call(
        body,
        grid=(N // BLK,),
        in_specs=[
            pl.BlockSpec((NC, BLK, D), lambda i: (0, i, 0)),
            pl.BlockSpec((BLK, D), lambda i: (i, 0)),
            pl.BlockSpec((D, dout), lambda i: (0, 0)),
            pl.BlockSpec((BLK, dout), lambda i: (i, 0)),
        ],
        out_specs=pl.BlockSpec((BLK, dout), lambda i: (i, 0)),
        out_shape=jax.ShapeDtypeStruct((N, dout), jnp.float32),
        input_output_aliases={3: 0},
    )(sp, dinv_bc, Wk, acc)


def kernel(x, edge_index, edge_attr, W1, b1, W2, b2, W3, b3):
    del edge_attr
    src = edge_index[0].reshape(NW, EPT)
    dst = edge_index[1].reshape(NW, NCHUNK, C)
    zeros = jnp.zeros((NPT, D), jnp.float32)

    degp = _deg_kernel(dst, zeros)
    dinv_bc = _dinv_call(degp)

    h = x
    for W, b, relu in ((W1, b1, True), (W2, b2, True), (W3, b3, False)):
        acc, u = _start_call(h, W[0], b, dinv_bc)
        for k in range(1, 4):
            sp = _hop_kernel(u, src, dst, zeros)
            if k < 3:
                acc, u = _mid_call(sp, dinv_bc, W[k], acc)
            else:
                h = _final_call(sp, dinv_bc, W[k], acc, relu)
    return h


# speedup vs baseline: 14.3732x; 1.2910x over previous
import functools

import jax
import jax.numpy as jnp
from jax import lax
from jax.experimental import pallas as pl
from jax.experimental.pallas import tpu as pltpu
from jax.experimental.pallas import tpu_sc as plsc

N = 10000
E = 320000
D = 128
NC = 2
NS = 16
NW = NC * NS
EPT = E // NW
C = 40
NCHUNK = EPT // C
DEPTH = 5
NROUND = NCHUNK // DEPTH
NP = 10240
NPT = NP // NS
BLK = 2000
BLKD = 2048
DEGW = 128


def _sc_mesh():
    return plsc.VectorSubcoreMesh(core_axis_name="c", subcore_axis_name="s",
                                  num_cores=NC, num_subcores=NS)


def _fill(buf, rows, width, value):
    vec = jnp.full((16,), value, jnp.float32)

    def body(r, _):
        for c8 in range(width // 16):
            buf[r, pl.ds(c8 * 16, 16)] = vec
        return 0

    lax.fori_loop(0, rows, body, 0)


def _deg_scratch():
    return [
        pltpu.VMEM((NCHUNK, C), jnp.int32),
        pltpu.VMEM((C, DEGW), jnp.float32),
        pltpu.VMEM_SHARED((NP, DEGW), jnp.float32),
    ] + [pltpu.SemaphoreType.DMA] * DEPTH


def _deg_kernel_body(dst_hbm, zeros_hbm, out_hbm, dst_v, buf, acc, *sems):
    cid = lax.axis_index("c")
    sid = lax.axis_index("s")
    wid = cid * NS + sid
    pltpu.sync_copy(dst_hbm.at[wid], dst_v)
    pltpu.sync_copy(zeros_hbm, acc.at[pl.ds(sid * NPT, NPT)])
    _fill(buf, C, DEGW, 1.0)
    plsc.subcore_barrier()

    def rnd(k, _):
        base = k * DEPTH
        for s in range(DEPTH):
            pltpu.async_copy(buf, acc.at[dst_v.at[base + s]], sems[s],
                             add=True)
        for s in range(DEPTH):
            pltpu.make_async_copy(buf, acc.at[dst_v.at[base + s]],
                                  sems[s]).wait()
        return 0

    lax.fori_loop(0, NROUND, rnd, 0)
    plsc.subcore_barrier()
    pltpu.sync_copy(acc.at[pl.ds(sid * NPT, NPT)],
                    out_hbm.at[cid, pl.ds(sid * NPT, NPT)])


_deg_kernel = pl.kernel(
    _deg_kernel_body,
    out_type=jax.ShapeDtypeStruct((NC, NP, DEGW), jnp.float32),
    mesh=_sc_mesh(),
    scratch_types=_deg_scratch(),
)


def _hop_scratch():
    return (
        [
            pltpu.VMEM((EPT,), jnp.int32),
            pltpu.VMEM((EPT,), jnp.int32),
        ]
        + [pltpu.VMEM((C, D), jnp.float32)] * DEPTH
        + [pltpu.VMEM_SHARED((NP, D), jnp.float32)]
        + [pltpu.SemaphoreType.DMA] * (2 * DEPTH)
    )


def _hop_kernel_body(u_hbm, src_hbm, dst_hbm, zeros_hbm, out_hbm,
                     src_v, dst_v, *rest):
    bufs = rest[:DEPTH]
    acc = rest[DEPTH]
    gsems = rest[DEPTH + 1:2 * DEPTH + 1]
    ssems = rest[2 * DEPTH + 1:]
    cid = lax.axis_index("c")
    sid = lax.axis_index("s")
    wid = cid * NS + sid
    pltpu.sync_copy(src_hbm.at[wid], src_v)
    pltpu.sync_copy(dst_hbm.at[wid], dst_v)
    pltpu.sync_copy(zeros_hbm, acc.at[pl.ds(sid * NPT, NPT)])
    plsc.subcore_barrier()

    def gather(j, s):
        pltpu.async_copy(u_hbm.at[src_v.at[pl.ds(j * C, C)]], bufs[s],
                         gsems[s])

    def wait_gather(j, s):
        pltpu.make_async_copy(u_hbm.at[src_v.at[pl.ds(j * C, C)]], bufs[s],
                              gsems[s]).wait()

    def scatter(j, s):
        pltpu.async_copy(bufs[s], acc.at[dst_v.at[pl.ds(j * C, C)]],
                         ssems[s], add=True)

    def wait_scatter(j, s):
        pltpu.make_async_copy(bufs[s], acc.at[dst_v.at[pl.ds(j * C, C)]],
                              ssems[s]).wait()

    for s in range(DEPTH):
        gather(s, s)

    def rnd(k, _):
        base = k * DEPTH
        for s in range(DEPTH):
            wait_gather(base + s, s)
            scatter(base + s, s)
        for s in range(DEPTH):
            wait_scatter(base + s, s)

            @pl.when(base + s + DEPTH < NCHUNK)
            def _(s=s, j=base + s + DEPTH):
                gather(j, s)

        return 0

    lax.fori_loop(0, NROUND, rnd, 0)
    plsc.subcore_barrier()
    pltpu.sync_copy(acc.at[pl.ds(sid * NPT, NPT)],
                    out_hbm.at[cid, pl.ds(sid * NPT, NPT)])


_hop_kernel = pl.kernel(
    _hop_kernel_body,
    out_type=jax.ShapeDtypeStruct((NC, NP, D), jnp.float32),
    mesh=_sc_mesh(),
    scratch_types=_hop_scratch(),
)


def _dinv_call(degp):
    def body(d_ref, o_ref):
        d = d_ref[0] + d_ref[1]
        d0 = d[:, 0:1]
        dinv = jnp.where(d0 > 0, lax.rsqrt(d0), 0.0)
        o_ref[...] = jnp.broadcast_to(dinv, (BLKD, D))

    return pl.pallas_call(
        body,
        grid=(NP // BLKD,),
        in_specs=[pl.BlockSpec((NC, BLKD, DEGW), lambda i: (0, i, 0))],
        out_specs=pl.BlockSpec((BLKD, D), lambda i: (i, 0)),
        out_shape=jax.ShapeDtypeStruct((NP, D), jnp.float32),
    )(degp)


def _start_call(h, W0, b, dinv_bc):
    dout = W0.shape[1]

    def body(h_ref, w_ref, b_ref, dv_ref, acc_ref, u_ref):
        hh = h_ref[...]
        acc_ref[...] = (
            jnp.dot(hh, w_ref[...], preferred_element_type=jnp.float32)
            + b_ref[...]
        )
        u_ref[...] = dv_ref[...] * hh

    return pl.pallas_call(
        body,
        grid=(N // BLK,),
        in_specs=[
            pl.BlockSpec((BLK, D), lambda i: (i, 0)),
            pl.BlockSpec((D, dout), lambda i: (0, 0)),
            pl.BlockSpec((1, dout), lambda i: (0, 0)),
            pl.BlockSpec((BLK, D), lambda i: (i, 0)),
        ],
        out_specs=[
            pl.BlockSpec((BLK, dout), lambda i: (i, 0)),
            pl.BlockSpec((BLK, D), lambda i: (i, 0)),
        ],
        out_shape=[
            jax.ShapeDtypeStruct((N, dout), jnp.float32),
            jax.ShapeDtypeStruct((N, D), jnp.float32),
        ],
    )(h, W0, b.reshape(1, dout), dinv_bc)


def _mid_call(sp, dinv_bc, Wk, acc):
    dout = Wk.shape[1]

    def body(sp_ref, dv_ref, w_ref, acc_ref, accO_ref, u_ref):
        s = sp_ref[0] + sp_ref[1]
        dv = dv_ref[...]
        t = dv * s
        accO_ref[...] = acc_ref[...] + jnp.dot(
            t, w_ref[...], preferred_element_type=jnp.float32)
        u_ref[...] = dv * t

    return pl.pallas_call(
        body,
        grid=(N // BLK,),
        in_specs=[
            pl.BlockSpec((NC, BLK, D), lambda i: (0, i, 0)),
            pl.BlockSpec((BLK, D), lambda i: (i, 0)),
            pl.BlockSpec((D, dout), lambda i: (0, 0)),
            pl.BlockSpec((BLK, dout), lambda i: (i, 0)),
        ],
        out_specs=[
            pl.BlockSpec((BLK, dout), lambda i: (i, 0)),
            pl.BlockSpec((BLK, D), lambda i: (i, 0)),
        ],
        out_shape=[
            jax.ShapeDtypeStruct((N, dout), jnp.float32),
            jax.ShapeDtypeStruct((N, D), jnp.float32),
        ],
        input_output_aliases={3: 0},
    )(sp, dinv_bc, Wk, acc)


def _final_call(sp, dinv_bc, Wk, acc, relu):
    dout = Wk.shape[1]

    def body(sp_ref, dv_ref, w_ref, acc_ref, o_ref):
        s = sp_ref[0] + sp_ref[1]
        t = dv_ref[...] * s
        o = acc_ref[...] + jnp.dot(
            t, w_ref[...], preferred_element_type=jnp.float32)
        o_ref[...] = jnp.maximum(o, 0.0) if relu else o

    return pl.pallas_call(
        body,
        grid=(N // BLK,),
        in_specs=[
            pl.BlockSpec((NC, BLK, D), lambda i: (0, i, 0)),
            pl.BlockSpec((BLK, D), lambda i: (i, 0)),
            pl.BlockSpec((D, dout), lambda i: (0, 0)),
            pl.BlockSpec((BLK, dout), lambda i: (i, 0)),
        ],
        out_specs=pl.BlockSpec((BLK, dout), lambda i: (i, 0)),
        out_shape=jax.ShapeDtypeStruct((N, dout), jnp.float32),
        input_output_aliases={3: 0},
    )(sp, dinv_bc, Wk, acc)


def kernel(x, edge_index, edge_attr, W1, b1, W2, b2, W3, b3):
    del edge_attr
    src = edge_index[0].reshape(NW, EPT)
    dst = edge_index[1].reshape(NW, EPT)
    dstc = edge_index[1].reshape(NW, NCHUNK, C)
    zeros = jnp.zeros((NPT, D), jnp.float32)

    degp = _deg_kernel(dstc, zeros)
    dinv_bc = _dinv_call(degp)

    h = x
    for W, b, relu in ((W1, b1, True), (W2, b2, True), (W3, b3, False)):
        acc, u = _start_call(h, W[0], b, dinv_bc)
        for k in range(1, 4):
            sp = _hop_kernel(u, src, dst, zeros)
            if k < 3:
                acc, u = _mid_call(sp, dinv_bc, W[k], acc)
            else:
                h = _final_call(sp, dinv_bc, W[k], acc, relu)
    return h
